# Initial kernel scaffold; baseline (speedup 1.0000x reference)
import jax
import jax.numpy as jnp
from jax.experimental import pallas as pl


def kernel(emb, pe_table):
    raise NotImplementedError("write your pallas kernel here")


# speedup vs baseline: 4.0174x; 4.0174x over previous
import jax
import jax.numpy as jnp
from jax.experimental import pallas as pl


def _pe_add_kernel(emb_ref, pe_ref, out_ref):
    pe = pe_ref[...]
    out_ref[...] = emb_ref[...] + pe[:, None, :]


def kernel(emb, pe_table):
    S, B, D = emb.shape
    SBLK = 512
    return pl.pallas_call(
        _pe_add_kernel,
        grid=(S // SBLK,),
        in_specs=[
            pl.BlockSpec((SBLK, B, D), lambda i: (i, 0, 0)),
            pl.BlockSpec((SBLK, D), lambda i: (i, 0)),
        ],
        out_specs=pl.BlockSpec((SBLK, B, D), lambda i: (i, 0, 0)),
        out_shape=jax.ShapeDtypeStruct((S, B, D), emb.dtype),
    )(emb, pe_table)


# speedup vs baseline: 4.1367x; 1.0297x over previous
import jax
import jax.numpy as jnp
from jax.experimental import pallas as pl


def _pe_add_kernel(emb_ref, pe_ref, out_ref):
    pe = pe_ref[...]
    out_ref[...] = emb_ref[...] + pe[:, None, :]


def kernel(emb, pe_table):
    S, B, D = emb.shape
    SBLK = 1024
    return pl.pallas_call(
        _pe_add_kernel,
        grid=(S // SBLK,),
        in_specs=[
            pl.BlockSpec((SBLK, B, D), lambda i: (i, 0, 0)),
            pl.BlockSpec((SBLK, D), lambda i: (i, 0)),
        ],
        out_specs=pl.BlockSpec((SBLK, B, D), lambda i: (i, 0, 0)),
        out_shape=jax.ShapeDtypeStruct((S, B, D), emb.dtype),
    )(emb, pe_table)
